# Initial kernel scaffold; baseline (speedup 1.0000x reference)
import jax
import jax.numpy as jnp
from jax.experimental import pallas as pl


def kernel(input, edge_sources, edge_targets, W):
    raise NotImplementedError("write your pallas kernel here")


# speedup vs baseline: 7.9765x; 7.9765x over previous
import functools

import jax
import jax.numpy as jnp
from jax import lax
from jax.experimental import pallas as pl
from jax.experimental.pallas import tpu as pltpu
from jax.experimental.pallas import tpu_sc as plsc

NUM_CORES = 2
NUM_SUBCORES = 16
NUM_TILES = NUM_CORES * NUM_SUBCORES
CHUNK = 80


def _messages(x, wt, dout, bm):
    n, din = x.shape

    def body(x_ref, wt_ref, m_ref):
        e = jnp.dot(x_ref[...], wt_ref[...], preferred_element_type=jnp.float32)
        m_ref[...] = jax.nn.sigmoid(e[:, :dout]) * e[:, dout:]

    return pl.pallas_call(
        body,
        grid=(n // bm,),
        in_specs=[
            pl.BlockSpec((bm, din), lambda i: (i, 0)),
            pl.BlockSpec((din, 2 * dout), lambda i: (0, 0)),
        ],
        out_specs=pl.BlockSpec((bm, dout), lambda i: (i, 0)),
        out_shape=jax.ShapeDtypeStruct((n, dout), jnp.float32),
    )(x, wt)


def _sc_scatter(m, src_r, tgt_r, zeros, n_chunks):
    n, d = zeros.shape
    rows_per_sub = n // NUM_SUBCORES

    def body(m_hbm, src_hbm, tgt_hbm, zero_hbm, out_hbm,
             idx_t, idx_s, rows, acc, sem):
        c = lax.axis_index("c")
        s = lax.axis_index("s")
        wid = c * NUM_SUBCORES + s

        pltpu.sync_copy(zero_hbm.at[pl.ds(s * rows_per_sub, rows_per_sub)],
                        acc.at[pl.ds(s * rows_per_sub, rows_per_sub)])
        pltpu.sync_copy(tgt_hbm.at[wid], idx_t)
        pltpu.sync_copy(src_hbm.at[wid], idx_s)
        plsc.subcore_barrier()

        def step(i, carry):
            pltpu.async_copy(m_hbm.at[idx_t.at[i]], rows, sem).wait()
            pltpu.sync_copy(rows, acc.at[idx_s.at[i]], add=True)
            return carry

        lax.fori_loop(0, n_chunks, step, 0)
        plsc.subcore_barrier()

        pltpu.sync_copy(acc.at[pl.ds(s * rows_per_sub, rows_per_sub)],
                        out_hbm.at[c, pl.ds(s * rows_per_sub, rows_per_sub)])

    fn = pl.kernel(
        body,
        out_type=jax.ShapeDtypeStruct((NUM_CORES, n, d), jnp.float32),
        mesh=plsc.VectorSubcoreMesh(core_axis_name="c", subcore_axis_name="s"),
        scratch_types=[
            pltpu.VMEM((n_chunks, CHUNK), jnp.int32),
            pltpu.VMEM((n_chunks, CHUNK), jnp.int32),
            pltpu.VMEM((CHUNK, d), jnp.float32),
            pltpu.VMEM_SHARED((n, d), jnp.float32),
            pltpu.SemaphoreType.DMA,
        ],
    )
    return fn(m, src_r, tgt_r, zeros)


def _combine(x, a0, a1, bm):
    n, d = x.shape

    def body(x_ref, a_ref, b_ref, o_ref):
        o_ref[...] = x_ref[...] + a_ref[...] + b_ref[...]

    spec = pl.BlockSpec((bm, d), lambda i: (i, 0))
    return pl.pallas_call(
        body,
        grid=(n // bm,),
        in_specs=[spec, spec, spec],
        out_specs=spec,
        out_shape=jax.ShapeDtypeStruct((n, d), jnp.float32),
    )(x, a0, a1)


def kernel(input, edge_sources, edge_targets, W):
    x = input
    n, din = x.shape
    dout = W.shape[0] // 2
    n_edges = edge_sources.shape[0]

    m = _messages(x, W.T, dout, bm=1000)

    group = NUM_TILES * CHUNK
    n_pad = (-n_edges) % group
    src = edge_sources.astype(jnp.int32)
    tgt = edge_targets.astype(jnp.int32)
    if n_pad:
        m_g = jnp.concatenate([m, jnp.zeros((8, dout), jnp.float32)], axis=0)
        src = jnp.concatenate([src, jnp.zeros((n_pad,), jnp.int32)])
        tgt = jnp.concatenate([tgt, jnp.full((n_pad,), n, jnp.int32)])
    else:
        m_g = m
    per_tile = (n_edges + n_pad) // NUM_TILES
    n_chunks = per_tile // CHUNK
    src_r = src.reshape(NUM_TILES, n_chunks, CHUNK)
    tgt_r = tgt.reshape(NUM_TILES, n_chunks, CHUNK)

    n_acc = ((n + 127) // 128) * 128
    zeros = jnp.zeros((n_acc, dout), jnp.float32)
    accs = _sc_scatter(m_g, src_r, tgt_r, zeros, n_chunks)

    return _combine(x, accs[0, :n, :], accs[1, :n, :], bm=1000)
